# Initial kernel scaffold; baseline (speedup 1.0000x reference)
import jax
import jax.numpy as jnp
from jax.experimental import pallas as pl


def kernel(x, weight):
    raise NotImplementedError("write your pallas kernel here")


# speedup vs baseline: 1.1032x; 1.1032x over previous
import functools

import jax
import jax.numpy as jnp
from jax import lax
from jax.experimental import pallas as pl
from jax.experimental.pallas import tpu as pltpu
from jax.experimental.pallas import tpu_sc as plsc

VOCAB = 1000000
EMBED_DIM = 32
BATCH = 16384
HIST = 50

_B = BATCH * HIST
_NC, _NS = 2, 16
_NW = _NC * _NS
_PER_W = _B // _NW
_CHUNK = 128
_CHUNKS_W = _PER_W // _CHUNK
_K = 8
_MACRO = _CHUNKS_W // _K
_ROWS_BLK = _K * _CHUNK


def _embed_kernel(table_hbm, idx_hbm, out_hbm, idx_v, rows_v, gsem):
  wid = lax.axis_index("s") * _NC + lax.axis_index("c")
  pltpu.sync_copy(idx_hbm.at[pl.ds(wid * _CHUNKS_W, _CHUNKS_W)], idx_v)
  row_base = wid * _PER_W

  @pl.loop(0, _MACRO)
  def _macro(m):
    descs = []
    for j in range(_K):
      descs.append(
          pltpu.async_copy(
              table_hbm.at[idx_v.at[m * _K + j]],
              rows_v.at[pl.ds(j * _CHUNK, _CHUNK)],
              gsem,
          )
      )
    for d in descs:
      d.wait()
    pltpu.sync_copy(
        rows_v, out_hbm.at[pl.ds(row_base + m * _ROWS_BLK, _ROWS_BLK)]
    )


@jax.jit
def _embed(weight, idx2d):
  mesh = plsc.VectorSubcoreMesh(core_axis_name="c", subcore_axis_name="s")
  run = pl.kernel(
      _embed_kernel,
      out_type=jax.ShapeDtypeStruct((_B, EMBED_DIM), jnp.float32),
      mesh=mesh,
      scratch_types=[
          pltpu.VMEM((_CHUNKS_W, _CHUNK), jnp.int32),
          pltpu.VMEM((_ROWS_BLK, EMBED_DIM), jnp.float32),
          pltpu.SemaphoreType.DMA,
      ],
      compiler_params=pltpu.CompilerParams(use_tc_tiling_on_sc=False),
  )
  return run(weight, idx2d)


def kernel(x, weight):
  idx2d = x.reshape(-1).astype(jnp.int32).reshape(_B // _CHUNK, _CHUNK)
  flat = _embed(weight, idx2d)
  return flat.reshape(x.shape + (EMBED_DIM,))


# speedup vs baseline: 1.1143x; 1.0100x over previous
import functools

import jax
import jax.numpy as jnp
from jax import lax
from jax.experimental import pallas as pl
from jax.experimental.pallas import tpu as pltpu
from jax.experimental.pallas import tpu_sc as plsc

VOCAB = 1000000
EMBED_DIM = 32
BATCH = 16384
HIST = 50

_B = BATCH * HIST
_NC, _NS = 2, 16
_NW = _NC * _NS
_PER_W = _B // _NW
_CHUNK = 128
_CHUNKS_W = _PER_W // _CHUNK
_K = 4
_BUF_ROWS = _K * _CHUNK
_NBUF = 5
_MACRO = _CHUNKS_W // _K


def _embed_kernel(table_hbm, idx_hbm, out_hbm, idx_v, rows_v, gsem, osem):
  wid = lax.axis_index("s") * _NC + lax.axis_index("c")
  pltpu.sync_copy(idx_hbm.at[pl.ds(wid * _CHUNKS_W, _CHUNKS_W)], idx_v)
  row_base = wid * _PER_W

  def fire_gathers(m, b):
    for j in range(_K):
      pltpu.async_copy(
          table_hbm.at[idx_v.at[m * _K + j]],
          rows_v.at[b].at[pl.ds(j * _CHUNK, _CHUNK)],
          gsem.at[b],
      )

  def drain_gathers(b):
    for j in range(_K):
      pltpu.make_async_copy(
          table_hbm.at[idx_v.at[j]],
          rows_v.at[b].at[pl.ds(j * _CHUNK, _CHUNK)],
          gsem.at[b],
      ).wait()

  def fire_out(m, b):
    pltpu.async_copy(
        rows_v.at[b],
        out_hbm.at[pl.ds(row_base + m * _BUF_ROWS, _BUF_ROWS)],
        osem.at[b],
    )

  def wait_out(b):
    pltpu.make_async_copy(
        rows_v.at[b],
        out_hbm.at[pl.ds(0, _BUF_ROWS)],
        osem.at[b],
    ).wait()

  for g in range(_NBUF - 1):
    fire_gathers(g, g)

  @pl.loop(0, _MACRO, step=_NBUF)
  def _ring(m0):
    for b0 in range(_NBUF):
      m = m0 + b0
      b = b0
      nb = (b0 + _NBUF - 1) % _NBUF
      @pl.when(m + _NBUF - 1 < _MACRO)
      def _():
        @pl.when(m > 0)
        def _():
          wait_out(nb)
        fire_gathers(m + _NBUF - 1, nb)
      drain_gathers(b)
      fire_out(m, b)

  for b in range(_NBUF):
    wait_out(b)


@jax.jit
def _embed(weight, idx2d):
  mesh = plsc.VectorSubcoreMesh(core_axis_name="c", subcore_axis_name="s")
  run = pl.kernel(
      _embed_kernel,
      out_type=jax.ShapeDtypeStruct((_B, EMBED_DIM), jnp.float32),
      mesh=mesh,
      scratch_types=[
          pltpu.VMEM((_CHUNKS_W, _CHUNK), jnp.int32),
          pltpu.VMEM((_NBUF, _BUF_ROWS, EMBED_DIM), jnp.float32),
          pltpu.SemaphoreType.DMA((_NBUF,)),
          pltpu.SemaphoreType.DMA((_NBUF,)),
      ],
      compiler_params=pltpu.CompilerParams(use_tc_tiling_on_sc=False),
  )
  return run(weight, idx2d)


def kernel(x, weight):
  idx2d = x.reshape(-1).astype(jnp.int32).reshape(_B // _CHUNK, _CHUNK)
  flat = _embed(weight, idx2d)
  return flat.reshape(x.shape + (EMBED_DIM,))


# speedup vs baseline: 1.1148x; 1.0005x over previous
import functools

import jax
import jax.numpy as jnp
from jax import lax
from jax.experimental import pallas as pl
from jax.experimental.pallas import tpu as pltpu
from jax.experimental.pallas import tpu_sc as plsc

VOCAB = 1000000
EMBED_DIM = 32
BATCH = 16384
HIST = 50

_B = BATCH * HIST
_NC, _NS = 2, 16
_NW = _NC * _NS
_PER_W = _B // _NW
_CHUNK = 512
_NBUF = 5
_MACRO = _PER_W // _CHUNK


def _embed_kernel(table_hbm, idx_hbm, out_hbm, idx_v, rows_v, gsem, osem):
  wid = lax.axis_index("s") * _NC + lax.axis_index("c")
  pltpu.sync_copy(idx_hbm.at[pl.ds(wid * _PER_W, _PER_W)], idx_v)
  row_base = wid * _PER_W

  def fire_gather(m, b):
    pltpu.async_copy(
        table_hbm.at[idx_v.at[pl.ds(m * _CHUNK, _CHUNK)]],
        rows_v.at[b],
        gsem.at[b],
    )

  def drain_gather(b):
    pltpu.make_async_copy(
        table_hbm.at[idx_v.at[pl.ds(0, _CHUNK)]],
        rows_v.at[b],
        gsem.at[b],
    ).wait()

  def fire_out(m, b):
    pltpu.async_copy(
        rows_v.at[b],
        out_hbm.at[pl.ds(row_base + m * _CHUNK, _CHUNK)],
        osem.at[b],
    )

  def wait_out(b):
    pltpu.make_async_copy(
        rows_v.at[b],
        out_hbm.at[pl.ds(0, _CHUNK)],
        osem.at[b],
    ).wait()

  for g in range(_NBUF - 1):
    fire_gather(g, g)

  @pl.loop(0, _MACRO, step=_NBUF)
  def _ring(m0):
    for b0 in range(_NBUF):
      m = m0 + b0
      nb = (b0 + _NBUF - 1) % _NBUF
      @pl.when(m + _NBUF - 1 < _MACRO)
      def _():
        @pl.when(m > 0)
        def _():
          wait_out(nb)
        fire_gather(m + _NBUF - 1, nb)
      drain_gather(b0)
      fire_out(m, b0)

  for b in range(_NBUF):
    wait_out(b)


@jax.jit
def _embed(weight, idx):
  mesh = plsc.VectorSubcoreMesh(core_axis_name="c", subcore_axis_name="s")
  run = pl.kernel(
      _embed_kernel,
      out_type=jax.ShapeDtypeStruct((_B, EMBED_DIM), jnp.float32),
      mesh=mesh,
      scratch_types=[
          pltpu.VMEM((_PER_W,), jnp.int32),
          pltpu.VMEM((_NBUF, _CHUNK, EMBED_DIM), jnp.float32),
          pltpu.SemaphoreType.DMA((_NBUF,)),
          pltpu.SemaphoreType.DMA((_NBUF,)),
      ],
      compiler_params=pltpu.CompilerParams(use_tc_tiling_on_sc=False),
  )
  return run(weight, idx)


def kernel(x, weight):
  idx = x.reshape(-1).astype(jnp.int32)
  flat = _embed(weight, idx)
  return flat.reshape(x.shape + (EMBED_DIM,))
